# baseline (speedup 1.0000x reference)
import jax
import jax.numpy as jnp
from jax import lax
from jax.experimental import pallas as pl
from jax.experimental.pallas import tpu as pltpu
from jax.experimental.pallas import tpu_sc as plsc

D = 64
SEQ = 200
LANES = 16
NC, NS = 2, 16
NW = NC * NS

CROWS = 128


def _enc_body(ids_hbm, table_hbm, pos_hbm, out_hbm, idx_v, rows_v, pos_v, sem):
    n = out_hbm.shape[0]
    rpw = n // NW
    nch = rpw // CROWS

    wid = lax.axis_index("s") * NC + lax.axis_index("c")
    pltpu.sync_copy(pos_hbm, pos_v)
    base_w = wid * rpw

    def chunk_body(i, carry):
        base = base_w + i * CROWS
        pltpu.sync_copy(ids_hbm.at[pl.ds(base, CROWS)], idx_v)
        pltpu.async_copy(table_hbm.at[idx_v], rows_v, sem).wait()
        p0 = lax.rem(i * CROWS, SEQ)

        def add_body(r, c2):
            p = p0 + r
            p = jnp.where(p >= SEQ, p - SEQ, p)
            for c in range(D // LANES):
                sl = pl.ds(c * LANES, LANES)
                rows_v[r, sl] = rows_v[r, sl] + pos_v[p, sl]
            return c2

        lax.fori_loop(0, CROWS, add_body, 0)
        pltpu.sync_copy(rows_v, out_hbm.at[pl.ds(base, CROWS)])
        return carry

    lax.fori_loop(0, nch, chunk_body, 0)


def kernel(input_ids, embedding, positional_encoding):
    b, s = input_ids.shape
    ids_flat = input_ids.reshape(-1).astype(jnp.int32)
    mesh = plsc.VectorSubcoreMesh(core_axis_name="c", subcore_axis_name="s")
    out = pl.kernel(
        _enc_body,
        out_type=jax.ShapeDtypeStruct((b * s, D), jnp.float32),
        mesh=mesh,
        compiler_params=pltpu.CompilerParams(use_tc_tiling_on_sc=False),
        scratch_types=[
            pltpu.VMEM((CROWS,), jnp.int32),
            pltpu.VMEM((CROWS, D), jnp.float32),
            pltpu.VMEM((SEQ, D), jnp.float32),
            pltpu.SemaphoreType.DMA,
        ],
    )(ids_flat, embedding, positional_encoding)
    return out.reshape(b, s, D)


# speedup vs baseline: 1.4615x; 1.4615x over previous
import jax
import jax.numpy as jnp
from jax import lax
from jax.experimental import pallas as pl
from jax.experimental.pallas import tpu as pltpu
from jax.experimental.pallas import tpu_sc as plsc

D = 64
SEQ = 200
NC, NS = 2, 16
NW = NC * NS

CROWS = SEQ
NBUF = 4


def _enc_body(ids_hbm, table_hbm, pos_hbm, out_hbm,
              idx_v, rows_v, pos_sh, gsem, osem):
    n = out_hbm.shape[0]
    rpw = n // NW
    nch = rpw // CROWS

    sid = lax.axis_index("s")
    wid = sid * NC + lax.axis_index("c")

    @pl.when(sid == 0)
    def _stage_pos():
        pltpu.sync_copy(pos_hbm, pos_sh)

    plsc.subcore_barrier()
    base_w = wid * rpw

    def group_body(g, carry):
        for b in range(NBUF):
            c = g * NBUF + b

            @pl.when(jnp.logical_and(c >= NBUF, c < nch))
            def _drain_out():
                pltpu.make_async_copy(
                    rows_v.at[b],
                    out_hbm.at[pl.ds(base_w + (c - NBUF) * CROWS, CROWS)],
                    osem.at[b],
                ).wait()

            @pl.when(c < nch)
            def _issue():
                pltpu.sync_copy(pos_sh, rows_v.at[b])
                pltpu.sync_copy(ids_hbm.at[pl.ds(base_w + c * CROWS, CROWS)],
                                idx_v.at[b])
                pltpu.async_copy(table_hbm.at[idx_v.at[b]], rows_v.at[b],
                                 gsem.at[b], add=True)

            d = c - (NBUF - 1)
            b2 = (b + 1) % NBUF

            @pl.when(jnp.logical_and(d >= 0, d < nch))
            def _complete():
                pltpu.make_async_copy(table_hbm.at[idx_v.at[b2]],
                                      rows_v.at[b2], gsem.at[b2]).wait()
                pltpu.async_copy(
                    rows_v.at[b2],
                    out_hbm.at[pl.ds(base_w + d * CROWS, CROWS)],
                    osem.at[b2],
                )
        return carry

    lax.fori_loop(0, nch // NBUF + 1, group_body, 0)

    for b in range(NBUF):
        c_last = nch - NBUF + b
        pltpu.make_async_copy(
            rows_v.at[b],
            out_hbm.at[pl.ds(base_w + c_last * CROWS, CROWS)],
            osem.at[b],
        ).wait()


def kernel(input_ids, embedding, positional_encoding):
    b, s = input_ids.shape
    ids_flat = input_ids.reshape(-1).astype(jnp.int32)
    mesh = plsc.VectorSubcoreMesh(core_axis_name="c", subcore_axis_name="s")
    out = pl.kernel(
        _enc_body,
        out_type=jax.ShapeDtypeStruct((b * s, D), jnp.float32),
        mesh=mesh,
        compiler_params=pltpu.CompilerParams(use_tc_tiling_on_sc=False),
        scratch_types=[
            pltpu.VMEM((NBUF, CROWS), jnp.int32),
            pltpu.VMEM((NBUF, CROWS, D), jnp.float32),
            pltpu.VMEM_SHARED((SEQ, D), jnp.float32),
            pltpu.SemaphoreType.DMA((NBUF,)),
            pltpu.SemaphoreType.DMA((NBUF,)),
        ],
    )(ids_flat, embedding, positional_encoding)
    return out.reshape(b, s, D)
